# baseline (speedup 1.0000x reference)
import jax
import jax.numpy as jnp
from jax import lax
from jax.experimental import pallas as pl
from jax.experimental.pallas import tpu as pltpu
from jax.experimental.pallas import tpu_sc as plsc

N = 3276800
V = 1000000
NC = 2
NS = 16
NW = NC * NS
PER_W = N // NW
CHUNK = 4096
NCHUNK = PER_W // CHUNK
EPS = 1e-8
STAGE_PIECE = 10000
NPIECES = V // STAGE_PIECE

ROWS = 3200
COLS = 1024
BROWS = 320


def _sc_gather_body(idx_hbm, table_hbm, c_hbm,
                    idx_v, c_v, stage_v, tab_sh, semI, semG, semS):
    sid = lax.axis_index("s")
    wid = sid * NC + lax.axis_index("c")
    base = wid * PER_W

    for r in range((NPIECES + NS - 1) // NS):
        p = sid + r * NS

        @pl.when(p < NPIECES)
        def _():
            off = p * STAGE_PIECE
            pltpu.sync_copy(table_hbm.at[pl.ds(off, STAGE_PIECE)], stage_v)
            pltpu.sync_copy(stage_v, tab_sh.at[pl.ds(off, STAGE_PIECE)])

    plsc.subcore_barrier()

    loads = {}
    gathers = {}
    stores = {}

    def fire_idx(g):
        b = g % 2
        off = base + g * CHUNK
        loads[g] = pltpu.async_copy(
            idx_hbm.at[pl.ds(off, CHUNK)], idx_v[b], semI[b])

    fire_idx(0)
    if NCHUNK > 1:
        fire_idx(1)

    for g in range(NCHUNK):
        b = g % 2
        off = base + g * CHUNK
        if g >= 2:
            stores[g - 2].wait()
        loads[g].wait()
        gathers[g] = pltpu.async_copy(tab_sh.at[idx_v[b]], c_v[b], semG[b])
        gathers[g].wait()
        if g + 2 < NCHUNK:
            fire_idx(g + 2)
        stores[g] = pltpu.async_copy(
            c_v[b], c_hbm.at[pl.ds(off, CHUNK)], semS[b])

    for g in (NCHUNK - 2, NCHUNK - 1):
        if g >= 0:
            stores[g].wait()


def _tc_solve_body(consts_ref, hg_ref, hb_ref, dx_ref, c_ref, og_ref, ob_ref):
    a = consts_ref[0]
    b = consts_ref[1]
    lamR = consts_ref[2]
    lamb = consts_ref[3]
    lamd = consts_ref[4]
    dx = jnp.maximum(dx_ref[...], 1e-6)
    beta = b / (dx + EPS)
    alpha = a - beta
    c = c_ref[...]
    A = lamb + lamR * (alpha * alpha)
    B = lamR * (alpha * beta)
    C = lamd + lamR * (beta * beta)
    rhs1 = lamb * hg_ref[...] + lamR * alpha * c
    rhs2 = lamd * hb_ref[...] + lamR * beta * c
    denom = A * C - B * B + EPS
    og_ref[...] = (C * rhs1 - B * rhs2) / denom
    ob_ref[...] = (-B * rhs1 + A * rhs2) / denom


def kernel(hg_hat, hb_hat, dx, ghost_local_idx, a, b, lamR_raw, lamb_raw,
           lamd_raw, c_table):
    f32 = jnp.float32
    lamR = jax.nn.softplus(lamR_raw) + EPS
    lamb = jax.nn.softplus(lamb_raw) + EPS
    lamd = jax.nn.softplus(lamd_raw) + EPS
    consts = jnp.stack([a, b, lamR, lamb, lamd]).astype(f32)

    sc_gather = pl.kernel(
        _sc_gather_body,
        out_type=jax.ShapeDtypeStruct((N,), f32),
        mesh=plsc.VectorSubcoreMesh(core_axis_name="c", subcore_axis_name="s"),
        scratch_types=(
            (pltpu.VMEM((CHUNK,), jnp.int32),) * 2,
            (pltpu.VMEM((CHUNK,), f32),) * 2,
            pltpu.VMEM((STAGE_PIECE,), f32),
            pltpu.VMEM_SHARED((V,), f32),
            (pltpu.SemaphoreType.DMA,) * 2,
            (pltpu.SemaphoreType.DMA,) * 2,
            (pltpu.SemaphoreType.DMA,) * 2,
        ),
    )
    c_flat = sc_gather(
        ghost_local_idx.astype(jnp.int32),
        c_table.reshape(-1).astype(f32),
    )

    blk = pl.BlockSpec((BROWS, COLS), lambda i: (i, 0))
    outg, outb = pl.pallas_call(
        _tc_solve_body,
        grid=(ROWS // BROWS,),
        in_specs=[
            pl.BlockSpec(memory_space=pltpu.SMEM),
            blk, blk, blk, blk,
        ],
        out_specs=[blk, blk],
        out_shape=(
            jax.ShapeDtypeStruct((ROWS, COLS), f32),
            jax.ShapeDtypeStruct((ROWS, COLS), f32),
        ),
    )(
        consts,
        hg_hat.reshape(ROWS, COLS),
        hb_hat.reshape(ROWS, COLS),
        dx.reshape(ROWS, COLS),
        c_flat.reshape(ROWS, COLS),
    )
    return (outg.reshape(N, 1), outb.reshape(N, 1))


# speedup vs baseline: 3.5427x; 3.5427x over previous
import jax
import jax.numpy as jnp
from jax import lax
from jax.experimental import pallas as pl
from jax.experimental.pallas import tpu as pltpu
from jax.experimental.pallas import tpu_sc as plsc

N = 3276800
V = 1000000
NC = 2
NS = 16
NW = NC * NS
PER_W = N // NW
CHUNK = 4096
NCHUNK = PER_W // CHUNK
EPS = 1e-8
STAGE_PIECE = 10000
NPIECES = V // STAGE_PIECE

BLK = 327680


def _sc_gather_body(idx_hbm, table_hbm, c_hbm,
                    idx_v, c_v, stage_v, tab_sh, semI, semG, semS):
    sid = lax.axis_index("s")
    wid = sid * NC + lax.axis_index("c")
    base = wid * PER_W

    for r in range((NPIECES + NS - 1) // NS):
        p = sid + r * NS

        @pl.when(p < NPIECES)
        def _():
            off = p * STAGE_PIECE
            pltpu.sync_copy(table_hbm.at[pl.ds(off, STAGE_PIECE)], stage_v)
            pltpu.sync_copy(stage_v, tab_sh.at[pl.ds(off, STAGE_PIECE)])

    plsc.subcore_barrier()

    loads = {}
    gathers = {}
    stores = {}

    def fire_idx(g):
        b = g % 2
        off = base + g * CHUNK
        loads[g] = pltpu.async_copy(
            idx_hbm.at[pl.ds(off, CHUNK)], idx_v[b], semI[b])

    fire_idx(0)
    if NCHUNK > 1:
        fire_idx(1)

    for g in range(NCHUNK):
        b = g % 2
        off = base + g * CHUNK
        if g >= 2:
            stores[g - 2].wait()
        loads[g].wait()
        gathers[g] = pltpu.async_copy(tab_sh.at[idx_v[b]], c_v[b], semG[b])
        gathers[g].wait()
        if g + 2 < NCHUNK:
            fire_idx(g + 2)
        stores[g] = pltpu.async_copy(
            c_v[b], c_hbm.at[pl.ds(off, CHUNK)], semS[b])

    for g in (NCHUNK - 2, NCHUNK - 1):
        if g >= 0:
            stores[g].wait()


def _tc_solve_body(consts_ref, hg_ref, hb_ref, dx_ref, c_ref, og_ref, ob_ref):
    a = consts_ref[0]
    b = consts_ref[1]
    lamR = consts_ref[2]
    lamb = consts_ref[3]
    lamd = consts_ref[4]
    dx = jnp.maximum(dx_ref[...], 1e-6)
    beta = b / (dx + EPS)
    alpha = a - beta
    c = c_ref[...]
    A = lamb + lamR * (alpha * alpha)
    B = lamR * (alpha * beta)
    C = lamd + lamR * (beta * beta)
    rhs1 = lamb * hg_ref[...] + lamR * alpha * c
    rhs2 = lamd * hb_ref[...] + lamR * beta * c
    denom = A * C - B * B + EPS
    og_ref[...] = (C * rhs1 - B * rhs2) / denom
    ob_ref[...] = (-B * rhs1 + A * rhs2) / denom


def kernel(hg_hat, hb_hat, dx, ghost_local_idx, a, b, lamR_raw, lamb_raw,
           lamd_raw, c_table):
    f32 = jnp.float32
    lamR = jax.nn.softplus(lamR_raw) + EPS
    lamb = jax.nn.softplus(lamb_raw) + EPS
    lamd = jax.nn.softplus(lamd_raw) + EPS
    consts = jnp.stack([a, b, lamR, lamb, lamd]).astype(f32)

    sc_gather = pl.kernel(
        _sc_gather_body,
        out_type=jax.ShapeDtypeStruct((N,), f32),
        mesh=plsc.VectorSubcoreMesh(core_axis_name="c", subcore_axis_name="s"),
        scratch_types=(
            (pltpu.VMEM((CHUNK,), jnp.int32),) * 2,
            (pltpu.VMEM((CHUNK,), f32),) * 2,
            pltpu.VMEM((STAGE_PIECE,), f32),
            pltpu.VMEM_SHARED((V,), f32),
            (pltpu.SemaphoreType.DMA,) * 2,
            (pltpu.SemaphoreType.DMA,) * 2,
            (pltpu.SemaphoreType.DMA,) * 2,
        ),
    )
    c_flat = sc_gather(
        ghost_local_idx.astype(jnp.int32),
        c_table.reshape(-1).astype(f32),
    )

    blk = pl.BlockSpec((BLK,), lambda i: (i,))
    outg, outb = pl.pallas_call(
        _tc_solve_body,
        grid=(N // BLK,),
        in_specs=[
            pl.BlockSpec(memory_space=pltpu.SMEM),
            blk, blk, blk, blk,
        ],
        out_specs=[blk, blk],
        out_shape=(
            jax.ShapeDtypeStruct((N,), f32),
            jax.ShapeDtypeStruct((N,), f32),
        ),
    )(
        consts,
        hg_hat.reshape(N),
        hb_hat.reshape(N),
        dx.reshape(N),
        c_flat,
    )
    return (outg.reshape(N, 1), outb.reshape(N, 1))


# speedup vs baseline: 4.7589x; 1.3433x over previous
import jax
import jax.numpy as jnp
from jax import lax
from jax.experimental import pallas as pl
from jax.experimental.pallas import tpu as pltpu
from jax.experimental.pallas import tpu_sc as plsc

N = 3276800
V = 1000000
NC = 2
NS = 16
NW = NC * NS
PER_W = N // NW
CHUNK = 4096
NCHUNK = PER_W // CHUNK
EPS = 1e-8
STAGE_PIECE = 10000
NPIECES = V // STAGE_PIECE
VPAD = 1000448

BLK = 327680


def _sc_gather_body(idx_hbm, table_hbm, c_hbm,
                    idx_v, c_v, stage_v, tab_sh, semI, semG, semS):
    sid = lax.axis_index("s")
    wid = sid * NC + lax.axis_index("c")
    base = wid * PER_W

    for r in range((NPIECES + NS - 1) // NS):
        p = sid + r * NS

        @pl.when(p < NPIECES)
        def _():
            off = p * STAGE_PIECE
            pltpu.sync_copy(table_hbm.at[pl.ds(off, STAGE_PIECE)], stage_v)
            pltpu.sync_copy(stage_v, tab_sh.at[pl.ds(off, STAGE_PIECE)])

    plsc.subcore_barrier()

    loads = {}
    gathers = {}
    stores = {}

    def fire_idx(g):
        b = g % 2
        off = base + g * CHUNK
        loads[g] = pltpu.async_copy(
            idx_hbm.at[pl.ds(off, CHUNK)], idx_v[b], semI[b])

    fire_idx(0)
    if NCHUNK > 1:
        fire_idx(1)

    for g in range(NCHUNK):
        b = g % 2
        off = base + g * CHUNK
        if g >= 2:
            stores[g - 2].wait()
        loads[g].wait()
        gathers[g] = pltpu.async_copy(tab_sh.at[idx_v[b]], c_v[b], semG[b])
        gathers[g].wait()
        if g + 2 < NCHUNK:
            fire_idx(g + 2)
        stores[g] = pltpu.async_copy(
            c_v[b], c_hbm.at[pl.ds(off, CHUNK)], semS[b])

    for g in (NCHUNK - 2, NCHUNK - 1):
        if g >= 0:
            stores[g].wait()


def _tc_solve_body(consts_ref, hg_ref, hb_ref, dx_ref, c_ref, og_ref, ob_ref):
    a = consts_ref[0]
    b = consts_ref[1]
    lamR = consts_ref[2]
    lamb = consts_ref[3]
    lamd = consts_ref[4]
    dx = jnp.maximum(dx_ref[...], 1e-6)
    beta = b / (dx + EPS)
    alpha = a - beta
    c = c_ref[...]
    A = lamb + lamR * (alpha * alpha)
    B = lamR * (alpha * beta)
    C = lamd + lamR * (beta * beta)
    rhs1 = lamb * hg_ref[...] + lamR * alpha * c
    rhs2 = lamd * hb_ref[...] + lamR * beta * c
    denom = A * C - B * B + EPS
    og_ref[...] = (C * rhs1 - B * rhs2) / denom
    ob_ref[...] = (-B * rhs1 + A * rhs2) / denom


def kernel(hg_hat, hb_hat, dx, ghost_local_idx, a, b, lamR_raw, lamb_raw,
           lamd_raw, c_table):
    f32 = jnp.float32
    lamR = jax.nn.softplus(lamR_raw) + EPS
    lamb = jax.nn.softplus(lamb_raw) + EPS
    lamd = jax.nn.softplus(lamd_raw) + EPS
    consts = jnp.stack([a, b, lamR, lamb, lamd]).astype(f32)

    sc_gather = pl.kernel(
        _sc_gather_body,
        out_type=jax.ShapeDtypeStruct((N,), f32),
        mesh=plsc.VectorSubcoreMesh(core_axis_name="c", subcore_axis_name="s"),
        scratch_types=(
            (pltpu.VMEM((CHUNK,), jnp.int32),) * 2,
            (pltpu.VMEM((CHUNK,), f32),) * 2,
            pltpu.VMEM((STAGE_PIECE,), f32),
            pltpu.VMEM_SHARED((V,), f32),
            (pltpu.SemaphoreType.DMA,) * 2,
            (pltpu.SemaphoreType.DMA,) * 2,
            (pltpu.SemaphoreType.DMA,) * 2,
        ),
    )
    table_padded = jnp.pad(c_table.astype(f32), ((0, VPAD - V), (0, 0)))
    c_flat = sc_gather(
        ghost_local_idx.astype(jnp.int32),
        table_padded.reshape(VPAD),
    )

    blk = pl.BlockSpec((BLK,), lambda i: (i,))
    outg, outb = pl.pallas_call(
        _tc_solve_body,
        grid=(N // BLK,),
        in_specs=[
            pl.BlockSpec(memory_space=pltpu.SMEM),
            blk, blk, blk, blk,
        ],
        out_specs=[blk, blk],
        out_shape=(
            jax.ShapeDtypeStruct((N,), f32),
            jax.ShapeDtypeStruct((N,), f32),
        ),
    )(
        consts,
        hg_hat.reshape(N),
        hb_hat.reshape(N),
        dx.reshape(N),
        c_flat,
    )
    return (outg.reshape(N, 1), outb.reshape(N, 1))
